# baseline (speedup 1.0000x reference)
import functools

import jax
import jax.numpy as jnp
from jax import lax
from jax.experimental import pallas as pl
from jax.experimental.pallas import tpu as pltpu
from jax.experimental.pallas import tpu_sc as plsc

NUM_CLASSES = 1000
DIM = 128
BATCH = 16384

_info = plsc.get_sparse_core_info()
_NC, _NS = _info.num_cores, _info.num_subcores
_NW = _NC * _NS
_B_PER_W = BATCH // _NW
_NCHUNK = 2
_CHUNK = _B_PER_W // _NCHUNK


@functools.partial(
    pl.kernel,
    mesh=plsc.VectorSubcoreMesh(core_axis_name="c", subcore_axis_name="s"),
    out_type=jax.ShapeDtypeStruct((BATCH, DIM), jnp.float32),
    scratch_types=[
        pltpu.VMEM((_B_PER_W,), jnp.int32),
        pltpu.VMEM((_NCHUNK, _CHUNK, DIM), jnp.float32),
        pltpu.VMEM_SHARED((NUM_CLASSES + 1, DIM), jnp.float32),
        pltpu.SemaphoreType.DMA,
        pltpu.SemaphoreType.DMA,
    ],
)
def _gather_kernel(labels_hbm, table_hbm, out_hbm, idx_v, rows_v, tab_s, gsem, ssem):
    sid = lax.axis_index("s")
    wid = sid * _NC + lax.axis_index("c")
    base = wid * _B_PER_W
    @pl.when(sid == 0)
    def _():
        pltpu.sync_copy(table_hbm, tab_s)
    pltpu.sync_copy(labels_hbm.at[pl.ds(base, _B_PER_W)], idx_v)
    plsc.subcore_barrier()
    gathers = []
    for c in range(_NCHUNK):
        gathers.append(
            pltpu.async_copy(
                tab_s.at[idx_v.at[pl.ds(c * _CHUNK, _CHUNK)]],
                rows_v.at[c],
                gsem,
            )
        )
    stores = []
    for c in range(_NCHUNK):
        gathers[c].wait()
        stores.append(
            pltpu.async_copy(
                rows_v.at[c],
                out_hbm.at[pl.ds(base + c * _CHUNK, _CHUNK)],
                ssem,
            )
        )
    for s in stores:
        s.wait()


def kernel(labels, table):
    return _gather_kernel(labels.astype(jnp.int32), table)


# speedup vs baseline: 1.0102x; 1.0102x over previous
import functools

import jax
import jax.numpy as jnp
from jax import lax
from jax.experimental import pallas as pl
from jax.experimental.pallas import tpu as pltpu
from jax.experimental.pallas import tpu_sc as plsc

NUM_CLASSES = 1000
DIM = 128
BATCH = 16384

_info = plsc.get_sparse_core_info()
_NC, _NS = _info.num_cores, _info.num_subcores
_NW = _NC * _NS
_B_PER_W = BATCH // _NW
_NCHUNK = 4
_CHUNK = _B_PER_W // _NCHUNK
_RING = 2


@functools.partial(
    pl.kernel,
    mesh=plsc.VectorSubcoreMesh(core_axis_name="c", subcore_axis_name="s"),
    out_type=jax.ShapeDtypeStruct((BATCH, DIM), jnp.float32),
    scratch_types=[
        pltpu.VMEM((_B_PER_W,), jnp.int32),
        pltpu.VMEM((_RING, _CHUNK, DIM), jnp.float32),
        pltpu.VMEM_SHARED((NUM_CLASSES + 1, DIM), jnp.float32),
        pltpu.SemaphoreType.DMA,
        pltpu.SemaphoreType.DMA,
    ],
)
def _gather_kernel(labels_hbm, table_hbm, out_hbm, idx_v, rows_v, tab_s, gsem, ssem):
    sid = lax.axis_index("s")
    wid = sid * _NC + lax.axis_index("c")
    base = wid * _B_PER_W
    @pl.when(sid == 0)
    def _():
        pltpu.sync_copy(table_hbm, tab_s)
    pltpu.sync_copy(labels_hbm.at[pl.ds(base, _B_PER_W)], idx_v)
    plsc.subcore_barrier()
    gathers = [None] * _NCHUNK
    stores = [None] * _NCHUNK
    for c in range(_NCHUNK):
        if c >= _RING:
            stores[c - _RING].wait()
        gathers[c] = pltpu.async_copy(
            tab_s.at[idx_v.at[pl.ds(c * _CHUNK, _CHUNK)]],
            rows_v.at[c % _RING],
            gsem,
        )
        if c >= 1:
            gathers[c - 1].wait()
            stores[c - 1] = pltpu.async_copy(
                rows_v.at[(c - 1) % _RING],
                out_hbm.at[pl.ds(base + (c - 1) * _CHUNK, _CHUNK)],
                ssem,
            )
    gathers[_NCHUNK - 1].wait()
    stores[_NCHUNK - 1] = pltpu.async_copy(
        rows_v.at[(_NCHUNK - 1) % _RING],
        out_hbm.at[pl.ds(base + (_NCHUNK - 1) * _CHUNK, _CHUNK)],
        ssem,
    )
    stores[_NCHUNK - 2].wait()
    stores[_NCHUNK - 1].wait()


def kernel(labels, table):
    return _gather_kernel(labels.astype(jnp.int32), table)
